# baseline (speedup 1.0000x reference)
import functools

import jax
import jax.numpy as jnp
from jax import lax
from jax.experimental import pallas as pl
from jax.experimental.pallas import tpu as pltpu
from jax.experimental.pallas import tpu_sc as plsc

F32 = jnp.float32
I32 = jnp.int32
L = 16


def _mm_bias_kernel(x_ref, w_ref, b_ref, o_ref):
    o_ref[...] = (
        jnp.dot(x_ref[...], w_ref[...], preferred_element_type=F32) + b_ref[...]
    )


def _mm_bias(x, w, b, bm):
    m, k = x.shape
    n = w.shape[1]
    return pl.pallas_call(
        _mm_bias_kernel,
        grid=(m // bm,),
        in_specs=[
            pl.BlockSpec((bm, k), lambda i: (i, 0)),
            pl.BlockSpec((k, n), lambda i: (0, 0)),
            pl.BlockSpec((1, n), lambda i: (0, 0)),
        ],
        out_specs=pl.BlockSpec((bm, n), lambda i: (i, 0)),
        out_shape=jax.ShapeDtypeStruct((m, n), F32),
    )(x, w, b)


def _relu_mm_bias_kernel(x_ref, b0_ref, w_ref, b_ref, o_ref):
    x1 = jnp.maximum(x_ref[...] + b0_ref[...], 0.0)
    o_ref[...] = jnp.dot(x1, w_ref[...], preferred_element_type=F32) + b_ref[...]


def _relu_mm_bias(x, b0, w, b, bm):
    m, k = x.shape
    n = w.shape[1]
    return pl.pallas_call(
        _relu_mm_bias_kernel,
        grid=(m // bm,),
        in_specs=[
            pl.BlockSpec((bm, k), lambda i: (i, 0)),
            pl.BlockSpec((1, k), lambda i: (0, 0)),
            pl.BlockSpec((k, n), lambda i: (0, 0)),
            pl.BlockSpec((1, n), lambda i: (0, 0)),
        ],
        out_specs=pl.BlockSpec((bm, n), lambda i: (i, 0)),
        out_shape=jax.ShapeDtypeStruct((m, n), F32),
    )(x, b0, w, b)


def _gru_kernel(g2_ref, b2_ref, h_ref, wi_ref, bi_ref, wh_ref, bh_ref, o_ref):
    x2 = g2_ref[...] + b2_ref[...]
    h = h_ref[...]
    gi = jnp.dot(x2, wi_ref[...], preferred_element_type=F32) + bi_ref[...]
    gh = jnp.dot(h, wh_ref[...], preferred_element_type=F32) + bh_ref[...]
    d = h.shape[1]
    ir, iz, inn = gi[:, :d], gi[:, d:2 * d], gi[:, 2 * d:]
    hr, hz, hn = gh[:, :d], gh[:, d:2 * d], gh[:, 2 * d:]
    r = jax.nn.sigmoid(ir + hr)
    z = jax.nn.sigmoid(iz + hz)
    n = jnp.tanh(inn + r * hn)
    o_ref[...] = (1.0 - z) * n + z * h


def _gru(g2, b2, h, wiT, bi, whT, bh, bm):
    m, d = h.shape
    n3 = wiT.shape[1]
    return pl.pallas_call(
        _gru_kernel,
        grid=(m // bm,),
        in_specs=[
            pl.BlockSpec((bm, d), lambda i: (i, 0)),
            pl.BlockSpec((1, d), lambda i: (0, 0)),
            pl.BlockSpec((bm, d), lambda i: (i, 0)),
            pl.BlockSpec((d, n3), lambda i: (0, 0)),
            pl.BlockSpec((1, n3), lambda i: (0, 0)),
            pl.BlockSpec((d, n3), lambda i: (0, 0)),
            pl.BlockSpec((1, n3), lambda i: (0, 0)),
        ],
        out_specs=pl.BlockSpec((bm, d), lambda i: (i, 0)),
        out_shape=jax.ShapeDtypeStruct((m, d), F32),
    )(g2, b2, h, wiT, bi, whT, bh)


def _softplus(v):
    return jnp.maximum(v, 0.0) + jnp.log(1.0 + jnp.exp(-jnp.abs(v)))


def _heads_kernel(h_ref, wh_ref, bh_ref, w2_ref, b2_ref, wpd_ref, bpd_ref,
                  rw_ref, pd_ref, ap_ref):
    h = h_ref[0]
    hid = jnp.maximum(
        jnp.dot(h, wh_ref[...], preferred_element_type=F32) + bh_ref[...], 0.0)
    rw = jnp.dot(hid[:, :512], w2_ref[...], preferred_element_type=F32) + b2_ref[...]
    rw_ref[0] = jnp.concatenate(
        [rw[:, :2], jax.nn.sigmoid(rw[:, 2:])], axis=1)
    pd = _softplus(
        jnp.dot(hid[:, 512:], wpd_ref[...], preferred_element_type=F32)
        + bpd_ref[...])
    pd_ref[0] = pd
    ap_ref[...] = jnp.full((1, 8, 128), jnp.sum(pd[:, 0]) / h.shape[0], F32)


def _heads(h_new, whid, bhid, w2, b2, wpd, bpd):
    b, n, d = h_new.shape
    nh = whid.shape[1]
    return pl.pallas_call(
        _heads_kernel,
        grid=(b,),
        in_specs=[
            pl.BlockSpec((1, n, d), lambda i: (i, 0, 0)),
            pl.BlockSpec((d, nh), lambda i: (0, 0)),
            pl.BlockSpec((1, nh), lambda i: (0, 0)),
            pl.BlockSpec((512, 4), lambda i: (0, 0)),
            pl.BlockSpec((1, 4), lambda i: (0, 0)),
            pl.BlockSpec((256, 2), lambda i: (0, 0)),
            pl.BlockSpec((1, 2), lambda i: (0, 0)),
        ],
        out_specs=[
            pl.BlockSpec((1, n, 4), lambda i: (i, 0, 0)),
            pl.BlockSpec((1, n, 2), lambda i: (i, 0, 0)),
            pl.BlockSpec((1, 8, 128), lambda i: (i, 0, 0)),
        ],
        out_shape=[
            jax.ShapeDtypeStruct((b, n, 4), F32),
            jax.ShapeDtypeStruct((b, n, 2), F32),
            jax.ShapeDtypeStruct((b, 8, 128), F32),
        ],
    )(h_new, whid, bhid, w2, b2, wpd, bpd)


NTILES = 16


def _make_edge_call(NN, EB, K, C, R, NPH, S_t, CAPB, concat):
    H = K // C
    OUTW = 256
    HALF = NN // 2
    OUTP = NPH * R
    EPB = EB // L // NTILES
    BL = 48
    CH2 = 64
    mesh = plsc.VectorSubcoreMesh(core_axis_name="c", subcore_axis_name="s")

    def body(xl_hbm, xr_hbm, ed_hbm, att_hbm, we_hbm, outp_hbm,
             st_buf, cl, ch2, o_src, o_dst, o_ea, att_v, we_v,
             xlbufA, xrbufA, xlbufB, xrbufB, exbuf, acc, den, obuf,
             cntb, cntv, spm_l, spm_c, sem1, sem2, sem3, sem4):
        cid = lax.axis_index("c")
        tid = lax.axis_index("s")
        lanes = lax.broadcasted_iota(I32, (L,), 0)
        zf = jnp.zeros((L,), F32)
        zi = jnp.zeros((L,), I32)

        pltpu.sync_copy(att_hbm, att_v)
        pltpu.sync_copy(we_hbm, we_v)

        def _phase(p, _):
            lo = cid * HALF + p * R
            hi = cid * HALF + jnp.minimum(p * R + R, HALF)

            def _za(i, _):
                acc[pl.ds(i * L, L)] = zf
                return 0
            lax.fori_loop(0, (S_t + 1) * K // L, _za, 0)
            def _zd(i, _):
                den[pl.ds(i * L, L)] = zf
                return 0
            lax.fori_loop(0, (S_t * 4 + 32) // L, _zd, 0)

            def _s1chunk(ch, cnt):
                pltpu.sync_copy(
                    ed_hbm.at[pl.ds((tid * EPB + ch * 64) * BL, 64 * BL)],
                    st_buf)
                def _scan(i, cnt):
                    sv = st_buf[pl.ds(i * BL, L)]
                    dv = st_buf[pl.ds(i * BL + 16, L)]
                    ev = st_buf[pl.ds(i * BL + 32, L)]
                    m = (dv >= lo) & (dv < hi)
                    mi = m.astype(I32)
                    pos = cnt + plsc.cumsum(mi) - 1
                    pos = jnp.minimum(pos, CAPB * L - 1)
                    posb = lax.shift_right_logical(pos, 4) * BL + (pos & 15)
                    plsc.store_scatter(cl, [posb], sv, mask=m)
                    plsc.store_scatter(cl, [posb + 16], dv, mask=m)
                    plsc.store_scatter(cl, [posb + 32], ev, mask=m)
                    return cnt + jnp.sum(mi)
                return lax.fori_loop(0, 64, _scan, cnt)
            cnt = lax.fori_loop(0, EPB // 64, _s1chunk, jnp.int32(0))

            nblk = lax.shift_right_logical(cnt + 15, 4)
            def _pub(ci, _):
                pltpu.sync_copy(
                    cl.at[pl.ds(ci * 64 * BL, 64 * BL)],
                    spm_l.at[pl.ds((tid * CAPB + ci * 64) * BL, 64 * BL)])
                return 0
            lax.fori_loop(0, lax.shift_right_logical(nblk + 63, 6), _pub, 0)
            cntb[pl.ds(0, L)] = jnp.full((L,), cnt, I32)
            pltpu.sync_copy(cntb, spm_c.at[pl.ds(tid * L, L)])
            plsc.subcore_barrier()

            sub_lo = lo + tid * S_t
            sub_hi = jnp.minimum(sub_lo + S_t, hi)
            pltpu.sync_copy(spm_c, cntv)

            def _idx(jb):
                srcv = jnp.clip(o_src[pl.ds(jb * L, L)], 0, NN - 1)
                dstlv = o_dst[pl.ds(jb * L, L)]
                xrrows = jnp.clip(dstlv + sub_lo, 0, NN - 1)
                return srcv, dstlv, xrrows

            def _issue(jb, xlb, xrb, s1, s2):
                srcv, _, xrrows = _idx(jb)
                pltpu.async_copy(xl_hbm.at[srcv], xlb, s1)
                pltpu.async_copy(xr_hbm.at[xrrows], xrb, s2)

            def _wait(jb, xlb, xrb, s1, s2):
                srcv, _, xrrows = _idx(jb)
                pltpu.make_async_copy(xl_hbm.at[srcv], xlb, s1).wait()
                pltpu.make_async_copy(xr_hbm.at[xrrows], xrb, s2).wait()

            def _compute(jb, xlbuf, xrbuf):
                _, dstlv, _ = _idx(jb)
                eav = plsc.bitcast(o_ea[pl.ds(jb * L, L)], F32)
                for h in range(H):
                    def _c(cc, a):
                        c0 = h * C + cc * L
                        attv = att_v[pl.ds(c0, L)]
                        wev = we_v[pl.ds(c0, L)]
                        for j in range(L):
                            cv = jnp.full((L,), c0 + j, I32)
                            xlc = plsc.load_gather(xlbuf, [lanes, cv])
                            xrc = plsc.load_gather(xrbuf, [lanes, cv])
                            u = xlc + xrc + eav * wev[j]
                            a = a + jnp.maximum(u, 0.2 * u) * attv[j]
                        return a
                    av = lax.fori_loop(0, C // L, _c, zf)
                    exbuf[pl.ds(h * L, L)] = jnp.exp(av)
                exvs = [exbuf[pl.ds(hh * L, L)] for hh in range(H)]
                for e in range(L):
                    dstl_e = dstlv[e]
                    evec = zf
                    for h in range(H):
                        evec = evec + jnp.where(lanes == h, exvs[h][e], 0.0)
                    o4 = dstl_e * 4
                    den[pl.ds(o4, L)] = den[pl.ds(o4, L)] + evec
                    base = dstl_e * K
                    for h in range(H):
                        s = exvs[h][e]
                        def _p2(j, _):
                            o = h * C + j * L
                            acc[pl.ds(base + o, L)] = (
                                acc[pl.ds(base + o, L)]
                                + s * xlbuf[e, pl.ds(o, L)])
                            return 0
                        lax.fori_loop(0, C // L, _p2, 0)

            def _j(j, _):
                cntj = cntv[pl.ds(j * L, L)][0]
                nblkj = lax.shift_right_logical(cntj + 15, 4)
                def _chunk(ci, _):
                    pltpu.sync_copy(
                        spm_l.at[pl.ds((j * CAPB + ci * CH2) * BL, CH2 * BL)],
                        ch2)
                    def _scan2(i, c2):
                        gi = ci * CH2 * L + i * L
                        sv = ch2[pl.ds(i * BL, L)]
                        dv = ch2[pl.ds(i * BL + 16, L)]
                        ev = ch2[pl.ds(i * BL + 32, L)]
                        m = ((gi + lanes < cntj) & (dv >= sub_lo)
                             & (dv < sub_hi))
                        mi = m.astype(I32)
                        pos = c2 + plsc.cumsum(mi) - 1
                        plsc.store_scatter(o_src, [pos], sv, mask=m)
                        plsc.store_scatter(o_dst, [pos], dv - sub_lo, mask=m)
                        plsc.store_scatter(o_ea, [pos], ev, mask=m)
                        return c2 + jnp.sum(mi)
                    c2 = lax.fori_loop(0, CH2, _scan2, jnp.int32(0))
                    sent = jnp.full((L,), S_t, I32)
                    o_src[pl.ds(c2, L)] = lanes
                    o_dst[pl.ds(c2, L)] = sent
                    o_ea[pl.ds(c2, L)] = zi
                    o_src[pl.ds(c2 + L, L)] = lanes
                    o_dst[pl.ds(c2 + L, L)] = sent
                    o_ea[pl.ds(c2 + L, L)] = zi
                    nbp = lax.shift_right_logical(c2 + 31, 5)
                    @pl.when(nbp > 0)
                    def _():
                        _issue(0, xlbufA, xrbufA, sem1, sem2)
                    def _pair(jp, _):
                        b0 = jp * 2
                        _issue(b0 + 1, xlbufB, xrbufB, sem3, sem4)
                        _wait(b0, xlbufA, xrbufA, sem1, sem2)
                        _compute(b0, xlbufA, xrbufA)
                        _issue(b0 + 2, xlbufA, xrbufA, sem1, sem2)
                        _wait(b0 + 1, xlbufB, xrbufB, sem3, sem4)
                        _compute(b0 + 1, xlbufB, xrbufB)
                        return 0
                    lax.fori_loop(0, nbp, _pair, 0)
                    @pl.when(nbp > 0)
                    def _():
                        _wait(nbp * 2, xlbufA, xrbufA, sem1, sem2)
                    return 0
                lax.fori_loop(
                    0, lax.shift_right_logical(nblkj + CH2 - 1, 6), _chunk, 0)
                return 0
            lax.fori_loop(0, NTILES, _j, 0)

            outbase = cid * OUTP + p * R + tid * S_t
            def _wb(b, _):
                sv0 = 1.0 / (den[pl.ds(b * 32, L)] + 1e-16)
                sv1 = 1.0 / (den[pl.ds(b * 32 + 16, L)] + 1e-16)
                for rr in range(8):
                    sv = sv0 if rr < 4 else sv1
                    ri = rr if rr < 4 else rr - 4
                    for cj in range(OUTW // L):
                        o = cj * L
                        if concat:
                            s = sv[ri * 4 + o // C]
                            obuf[rr, pl.ds(o, L)] = s * acc[
                                pl.ds((b * 8 + rr) * K + o, L)]
                        else:
                            a = zf
                            for h in range(H):
                                s = sv[ri * 4 + h]
                                a = a + s * acc[
                                    pl.ds((b * 8 + rr) * K + h * C + o, L)]
                            obuf[rr, pl.ds(o, L)] = a * (1.0 / H)
                pltpu.sync_copy(obuf, outp_hbm.at[pl.ds(outbase + b * 8, 8)])
                return 0
            lax.fori_loop(0, S_t // 8, _wb, 0)
            plsc.subcore_barrier()
            return 0
        lax.fori_loop(0, NPH, _phase, 0)

    scratch = [
        pltpu.VMEM((64 * BL,), I32),
        pltpu.VMEM((CAPB * BL,), I32),
        pltpu.VMEM((CH2 * BL,), I32),
        pltpu.VMEM((CH2 * L + 48,), I32),
        pltpu.VMEM((CH2 * L + 48,), I32),
        pltpu.VMEM((CH2 * L + 48,), I32),
        pltpu.VMEM((K,), F32), pltpu.VMEM((K,), F32),
        pltpu.VMEM((L, K), F32), pltpu.VMEM((L, K), F32),
        pltpu.VMEM((L, K), F32), pltpu.VMEM((L, K), F32),
        pltpu.VMEM((H * L,), F32),
        pltpu.VMEM(((S_t + 1) * K,), F32),
        pltpu.VMEM((S_t * 4 + 32,), F32),
        pltpu.VMEM((8, OUTW), F32),
        pltpu.VMEM((L,), I32), pltpu.VMEM((NTILES * L,), I32),
        pltpu.VMEM_SHARED(((NTILES * CAPB + CH2) * BL,), I32),
        pltpu.VMEM_SHARED((NTILES * L,), I32),
        pltpu.SemaphoreType.DMA, pltpu.SemaphoreType.DMA,
        pltpu.SemaphoreType.DMA, pltpu.SemaphoreType.DMA,
    ]
    return functools.partial(
        pl.kernel, body, mesh=mesh,
        out_type=jax.ShapeDtypeStruct((2 * OUTP, OUTW), F32),
        compiler_params=pltpu.CompilerParams(needs_layout_passes=False),
        scratch_types=scratch)


def kernel(h, c_temp, c_stereo, e_proj, f_Lt, edges, edge_attr, params):
    p = params
    B, N, D = h.shape
    NN = B * N
    E = edges.shape[1]
    h2 = h.reshape(NN, D)

    x = jnp.concatenate(
        [c_temp.reshape(NN, -1), c_stereo.reshape(NN, -1),
         e_proj.reshape(NN, -1), f_Lt.reshape(NN, -1),
         jnp.zeros((NN, 126), F32)], axis=-1)
    w1 = jnp.concatenate([p['W1l'], p['W1r']], axis=1)
    w1 = jnp.concatenate([w1, jnp.zeros((126, 512), F32)], axis=0)
    b1 = jnp.concatenate([p['b1l'], p['b1r']]).reshape(1, 512)
    xlr1 = _mm_bias(x, w1, b1, 1000)
    xl1, xr1 = xlr1[:, :256], xlr1[:, 256:]

    EB = ((E + 255) // 256) * 256
    src = edges[0].astype(I32)
    dst = edges[1].astype(I32)
    eav = edge_attr[:, 0]
    srcp = jnp.concatenate([src, jnp.zeros((EB - E,), I32)])
    dstp = jnp.concatenate([dst, jnp.full((EB - E,), NN, I32)])
    eap = jnp.concatenate([eav, jnp.zeros((EB - E,), F32)])
    ed = jnp.concatenate(
        [srcp.reshape(-1, L), dstp.reshape(-1, L),
         lax.bitcast_convert_type(eap, I32).reshape(-1, L)],
        axis=1).reshape(-1)

    edge1 = _make_edge_call(NN, EB, 256, 64, 2560, 2, 160, 256, True)()
    o1 = edge1(xl1, xr1, ed, p['att1'].reshape(256), p['W1e'][0])
    g1 = jnp.concatenate([o1[0:NN // 2], o1[5120:5120 + NN // 2]])

    w2 = jnp.concatenate([p['W2l'], p['W2r']], axis=1)
    b2 = jnp.concatenate([p['b2l'], p['b2r']]).reshape(1, 2048)
    xlr2 = _relu_mm_bias(g1, p['bias1'].reshape(1, 256), w2, b2, 1000)
    xl2, xr2 = xlr2[:, :1024], xlr2[:, 1024:]

    edge2 = _make_edge_call(NN, EB, 1024, 256, 512, 10, 32, 96, False)()
    o2 = edge2(xl2, xr2, ed, p['att2'].reshape(1024), p['W2e'][0])
    g2 = jnp.concatenate([o2[0:NN // 2], o2[5120:5120 + NN // 2]])

    h_new_flat = _gru(g2, p['bias2'].reshape(1, 256), h2,
                      p['Wih'].T, p['bih'].reshape(1, 768),
                      p['Whh'].T, p['bhh'].reshape(1, 768), 1000)
    h_new = h_new_flat.reshape(B, N, D)

    whid = jnp.concatenate([p['Wr1'], p['Ww1'], p['Wp1'], p['Wd1']], axis=1)
    bhid = jnp.concatenate([p['br1'], p['bw1'], p['bp1'], p['bd1']]).reshape(1, 768)
    w2h = jnp.zeros((512, 4), F32)
    w2h = w2h.at[:256, :2].set(p['Wr2']).at[256:, 2:].set(p['Ww2'])
    b2h = jnp.concatenate([p['br2'], p['bw2']]).reshape(1, 4)
    wpd = jnp.zeros((256, 2), F32)
    wpd = wpd.at[:128, 0].set(p['Wp2'][:, 0]).at[128:, 1].set(p['Wd2'][:, 0])
    bpd = jnp.concatenate([p['bp2'], p['bd2']]).reshape(1, 2)
    rw, pd, ap = _heads(h_new, whid, bhid, w2h, b2h, wpd, bpd)

    return (h_new, rw[..., :2], rw[..., 2:], ap[:, 0, :1], pd[..., 1:2])


# speedup vs baseline: 2.4420x; 2.4420x over previous
import functools

import jax
import jax.numpy as jnp
from jax import lax
from jax.experimental import pallas as pl
from jax.experimental.pallas import tpu as pltpu
from jax.experimental.pallas import tpu_sc as plsc

F32 = jnp.float32
I32 = jnp.int32
L = 16


def _mm_bias_kernel(x_ref, w_ref, b_ref, o_ref):
    o_ref[...] = (
        jnp.dot(x_ref[...], w_ref[...], preferred_element_type=F32) + b_ref[...]
    )


def _mm_bias(x, w, b, bm):
    m, k = x.shape
    n = w.shape[1]
    return pl.pallas_call(
        _mm_bias_kernel,
        grid=(m // bm,),
        in_specs=[
            pl.BlockSpec((bm, k), lambda i: (i, 0)),
            pl.BlockSpec((k, n), lambda i: (0, 0)),
            pl.BlockSpec((1, n), lambda i: (0, 0)),
        ],
        out_specs=pl.BlockSpec((bm, n), lambda i: (i, 0)),
        out_shape=jax.ShapeDtypeStruct((m, n), F32),
    )(x, w, b)


def _relu_mm_bias_kernel(x_ref, b0_ref, w_ref, b_ref, o_ref):
    x1 = jnp.maximum(x_ref[...] + b0_ref[...], 0.0)
    o_ref[...] = jnp.dot(x1, w_ref[...], preferred_element_type=F32) + b_ref[...]


def _relu_mm_bias(x, b0, w, b, bm):
    m, k = x.shape
    n = w.shape[1]
    return pl.pallas_call(
        _relu_mm_bias_kernel,
        grid=(m // bm,),
        in_specs=[
            pl.BlockSpec((bm, k), lambda i: (i, 0)),
            pl.BlockSpec((1, k), lambda i: (0, 0)),
            pl.BlockSpec((k, n), lambda i: (0, 0)),
            pl.BlockSpec((1, n), lambda i: (0, 0)),
        ],
        out_specs=pl.BlockSpec((bm, n), lambda i: (i, 0)),
        out_shape=jax.ShapeDtypeStruct((m, n), F32),
    )(x, b0, w, b)


def _gru_kernel(g2_ref, b2_ref, h_ref, wi_ref, bi_ref, wh_ref, bh_ref, o_ref):
    x2 = g2_ref[...] + b2_ref[...]
    h = h_ref[...]
    gi = jnp.dot(x2, wi_ref[...], preferred_element_type=F32) + bi_ref[...]
    gh = jnp.dot(h, wh_ref[...], preferred_element_type=F32) + bh_ref[...]
    d = h.shape[1]
    ir, iz, inn = gi[:, :d], gi[:, d:2 * d], gi[:, 2 * d:]
    hr, hz, hn = gh[:, :d], gh[:, d:2 * d], gh[:, 2 * d:]
    r = jax.nn.sigmoid(ir + hr)
    z = jax.nn.sigmoid(iz + hz)
    n = jnp.tanh(inn + r * hn)
    o_ref[...] = (1.0 - z) * n + z * h


def _gru(g2, b2, h, wiT, bi, whT, bh, bm):
    m, d = h.shape
    n3 = wiT.shape[1]
    return pl.pallas_call(
        _gru_kernel,
        grid=(m // bm,),
        in_specs=[
            pl.BlockSpec((bm, d), lambda i: (i, 0)),
            pl.BlockSpec((1, d), lambda i: (0, 0)),
            pl.BlockSpec((bm, d), lambda i: (i, 0)),
            pl.BlockSpec((d, n3), lambda i: (0, 0)),
            pl.BlockSpec((1, n3), lambda i: (0, 0)),
            pl.BlockSpec((d, n3), lambda i: (0, 0)),
            pl.BlockSpec((1, n3), lambda i: (0, 0)),
        ],
        out_specs=pl.BlockSpec((bm, d), lambda i: (i, 0)),
        out_shape=jax.ShapeDtypeStruct((m, d), F32),
    )(g2, b2, h, wiT, bi, whT, bh)


def _softplus(v):
    return jnp.maximum(v, 0.0) + jnp.log(1.0 + jnp.exp(-jnp.abs(v)))


def _heads_kernel(h_ref, wh_ref, bh_ref, w2_ref, b2_ref, wpd_ref, bpd_ref,
                  rw_ref, pd_ref, ap_ref):
    h = h_ref[0]
    hid = jnp.maximum(
        jnp.dot(h, wh_ref[...], preferred_element_type=F32) + bh_ref[...], 0.0)
    rw = jnp.dot(hid[:, :512], w2_ref[...], preferred_element_type=F32) + b2_ref[...]
    rw_ref[0] = jnp.concatenate(
        [rw[:, :2], jax.nn.sigmoid(rw[:, 2:])], axis=1)
    pd = _softplus(
        jnp.dot(hid[:, 512:], wpd_ref[...], preferred_element_type=F32)
        + bpd_ref[...])
    pd_ref[0] = pd
    ap_ref[...] = jnp.full((1, 8, 128), jnp.sum(pd[:, 0]) / h.shape[0], F32)


def _heads(h_new, whid, bhid, w2, b2, wpd, bpd):
    b, n, d = h_new.shape
    nh = whid.shape[1]
    return pl.pallas_call(
        _heads_kernel,
        grid=(b,),
        in_specs=[
            pl.BlockSpec((1, n, d), lambda i: (i, 0, 0)),
            pl.BlockSpec((d, nh), lambda i: (0, 0)),
            pl.BlockSpec((1, nh), lambda i: (0, 0)),
            pl.BlockSpec((512, 4), lambda i: (0, 0)),
            pl.BlockSpec((1, 4), lambda i: (0, 0)),
            pl.BlockSpec((256, 2), lambda i: (0, 0)),
            pl.BlockSpec((1, 2), lambda i: (0, 0)),
        ],
        out_specs=[
            pl.BlockSpec((1, n, 4), lambda i: (i, 0, 0)),
            pl.BlockSpec((1, n, 2), lambda i: (i, 0, 0)),
            pl.BlockSpec((1, 8, 128), lambda i: (i, 0, 0)),
        ],
        out_shape=[
            jax.ShapeDtypeStruct((b, n, 4), F32),
            jax.ShapeDtypeStruct((b, n, 2), F32),
            jax.ShapeDtypeStruct((b, 8, 128), F32),
        ],
    )(h_new, whid, bhid, w2, b2, wpd, bpd)


NTILES = 16


def _make_edge_call(NN, EB, K, C, R, NPH, S_t, CAPB, concat):
    H = K // C
    OUTW = 256
    HALF = NN // 2
    OUTP = NPH * R
    EPB = EB // L // NTILES
    BL = 48
    CH2 = 64
    mesh = plsc.VectorSubcoreMesh(core_axis_name="c", subcore_axis_name="s")

    def body(xl_hbm, xr_hbm, ed_hbm, att_hbm, we_hbm, outp_hbm,
             st_buf, cl, ch2, o_src, o_dst, o_ea, att_v, we_v,
             xlbuf, xrbuf, exbuf, acc, den, obuf,
             cntb, cntv, spm_l, spm_c, sem1, sem2):
        cid = lax.axis_index("c")
        tid = lax.axis_index("s")
        lanes = lax.broadcasted_iota(I32, (L,), 0)
        zf = jnp.zeros((L,), F32)
        zi = jnp.zeros((L,), I32)

        pltpu.sync_copy(att_hbm, att_v)
        pltpu.sync_copy(we_hbm, we_v)

        def _phase(p, _):
            lo = cid * HALF + p * R
            hi = cid * HALF + jnp.minimum(p * R + R, HALF)

            def _za(i, _):
                acc[pl.ds(i * L, L)] = zf
                return 0
            lax.fori_loop(0, (S_t + 1) * K // L, _za, 0)
            def _zd(i, _):
                den[pl.ds(i * L, L)] = zf
                return 0
            lax.fori_loop(0, (S_t * 4 + 32) // L, _zd, 0)

            def _s1chunk(ch, cnt):
                pltpu.sync_copy(
                    ed_hbm.at[pl.ds((tid * EPB + ch * 64) * BL, 64 * BL)],
                    st_buf)
                def _scan(i, cnt):
                    sv = st_buf[pl.ds(i * BL, L)]
                    dv = st_buf[pl.ds(i * BL + 16, L)]
                    ev = st_buf[pl.ds(i * BL + 32, L)]
                    m = (dv >= lo) & (dv < hi)
                    mi = m.astype(I32)
                    pos = cnt + plsc.cumsum(mi) - 1
                    pos = jnp.minimum(pos, CAPB * L - 1)
                    posb = lax.shift_right_logical(pos, 4) * BL + (pos & 15)
                    plsc.store_scatter(cl, [posb], sv, mask=m)
                    plsc.store_scatter(cl, [posb + 16], dv, mask=m)
                    plsc.store_scatter(cl, [posb + 32], ev, mask=m)
                    return cnt + jnp.sum(mi)
                return lax.fori_loop(0, 64, _scan, cnt)
            cnt = lax.fori_loop(0, EPB // 64, _s1chunk, jnp.int32(0))

            nblk = lax.shift_right_logical(cnt + 15, 4)
            def _pub(ci, _):
                pltpu.sync_copy(
                    cl.at[pl.ds(ci * 64 * BL, 64 * BL)],
                    spm_l.at[pl.ds((tid * CAPB + ci * 64) * BL, 64 * BL)])
                return 0
            lax.fori_loop(0, lax.shift_right_logical(nblk + 63, 6), _pub, 0)
            cntb[pl.ds(0, L)] = jnp.full((L,), cnt, I32)
            pltpu.sync_copy(cntb, spm_c.at[pl.ds(tid * L, L)])
            plsc.subcore_barrier()

            sub_lo = lo + tid * S_t
            sub_hi = jnp.minimum(sub_lo + S_t, hi)
            pltpu.sync_copy(spm_c, cntv)

            def _batch(jb, _):
                srcv = jnp.clip(o_src[pl.ds(jb * L, L)], 0, NN - 1)
                dstlv = o_dst[pl.ds(jb * L, L)]
                eav = plsc.bitcast(o_ea[pl.ds(jb * L, L)], F32)
                cp1 = pltpu.async_copy(xl_hbm.at[srcv], xlbuf, sem1)
                xrrows = jnp.clip(dstlv + sub_lo, 0, NN - 1)
                cp2 = pltpu.async_copy(xr_hbm.at[xrrows], xrbuf, sem2)
                cp1.wait()
                cp2.wait()
                eax = [eav[ee] for ee in range(L)]
                for h in range(H):
                    def _c(cc, accs):
                        o = h * C + cc * L
                        attv = att_v[pl.ds(o, L)]
                        wev = we_v[pl.ds(o, L)]
                        out = []
                        for e in range(L):
                            u = (xlbuf[e, pl.ds(o, L)]
                                 + xrbuf[e, pl.ds(o, L)] + eax[e] * wev)
                            out.append(
                                accs[e] + jnp.maximum(u, 0.2 * u) * attv)
                        return tuple(out)
                    accs = lax.fori_loop(0, C // L, _c, (zf,) * L)
                    av = zf
                    for e in range(L):
                        av = av + jnp.where(lanes == e, jnp.sum(accs[e]), 0.0)
                    exbuf[pl.ds(h * L, L)] = jnp.exp(av)
                exvs = [exbuf[pl.ds(hh * L, L)] for hh in range(H)]
                for e in range(L):
                    dstl_e = dstlv[e]
                    evec = zf
                    for h in range(H):
                        evec = evec + jnp.where(lanes == h, exvs[h][e], 0.0)
                    o4 = dstl_e * 4
                    den[pl.ds(o4, L)] = den[pl.ds(o4, L)] + evec
                    base = dstl_e * K
                    for h in range(H):
                        s = exvs[h][e]
                        def _p2(j, _):
                            o = h * C + j * L
                            acc[pl.ds(base + o, L)] = (
                                acc[pl.ds(base + o, L)]
                                + s * xlbuf[e, pl.ds(o, L)])
                            return 0
                        lax.fori_loop(0, C // L, _p2, 0)
                return 0

            def _j(j, _):
                cntj = cntv[pl.ds(j * L, L)][0]
                nblkj = lax.shift_right_logical(cntj + 15, 4)
                def _chunk(ci, _):
                    pltpu.sync_copy(
                        spm_l.at[pl.ds((j * CAPB + ci * CH2) * BL, CH2 * BL)],
                        ch2)
                    def _scan2(i, c2):
                        gi = ci * CH2 * L + i * L
                        sv = ch2[pl.ds(i * BL, L)]
                        dv = ch2[pl.ds(i * BL + 16, L)]
                        ev = ch2[pl.ds(i * BL + 32, L)]
                        m = ((gi + lanes < cntj) & (dv >= sub_lo)
                             & (dv < sub_hi))
                        mi = m.astype(I32)
                        pos = c2 + plsc.cumsum(mi) - 1
                        plsc.store_scatter(o_src, [pos], sv, mask=m)
                        plsc.store_scatter(o_dst, [pos], dv - sub_lo, mask=m)
                        plsc.store_scatter(o_ea, [pos], ev, mask=m)
                        return c2 + jnp.sum(mi)
                    nsc = jnp.minimum(CH2, nblkj - ci * CH2)
                    c2 = lax.fori_loop(0, nsc, _scan2, jnp.int32(0))
                    o_src[pl.ds(c2, L)] = lanes
                    o_dst[pl.ds(c2, L)] = jnp.full((L,), S_t, I32)
                    o_ea[pl.ds(c2, L)] = zi
                    nb = lax.shift_right_logical(c2 + L - 1, 4)
                    lax.fori_loop(0, nb, _batch, 0)
                    return 0
                lax.fori_loop(
                    0, lax.shift_right_logical(nblkj + CH2 - 1, 6), _chunk, 0)
                return 0
            lax.fori_loop(0, NTILES, _j, 0)

            outbase = cid * OUTP + p * R + tid * S_t
            def _wb(b, _):
                sv0 = 1.0 / (den[pl.ds(b * 32, L)] + 1e-16)
                sv1 = 1.0 / (den[pl.ds(b * 32 + 16, L)] + 1e-16)
                for rr in range(8):
                    sv = sv0 if rr < 4 else sv1
                    ri = rr if rr < 4 else rr - 4
                    for cj in range(OUTW // L):
                        o = cj * L
                        if concat:
                            s = sv[ri * 4 + o // C]
                            obuf[rr, pl.ds(o, L)] = s * acc[
                                pl.ds((b * 8 + rr) * K + o, L)]
                        else:
                            a = zf
                            for h in range(H):
                                s = sv[ri * 4 + h]
                                a = a + s * acc[
                                    pl.ds((b * 8 + rr) * K + h * C + o, L)]
                            obuf[rr, pl.ds(o, L)] = a * (1.0 / H)
                pltpu.sync_copy(obuf, outp_hbm.at[pl.ds(outbase + b * 8, 8)])
                return 0
            lax.fori_loop(0, S_t // 8, _wb, 0)
            plsc.subcore_barrier()
            return 0
        lax.fori_loop(0, NPH, _phase, 0)

    scratch = [
        pltpu.VMEM((64 * BL,), I32),
        pltpu.VMEM((CAPB * BL,), I32),
        pltpu.VMEM((CH2 * BL,), I32),
        pltpu.VMEM((CH2 * L + 48,), I32),
        pltpu.VMEM((CH2 * L + 48,), I32),
        pltpu.VMEM((CH2 * L + 48,), I32),
        pltpu.VMEM((K,), F32), pltpu.VMEM((K,), F32),
        pltpu.VMEM((L, K), F32), pltpu.VMEM((L, K), F32),
        pltpu.VMEM((H * L,), F32),
        pltpu.VMEM(((S_t + 1) * K,), F32),
        pltpu.VMEM((S_t * 4 + 32,), F32),
        pltpu.VMEM((8, OUTW), F32),
        pltpu.VMEM((L,), I32), pltpu.VMEM((NTILES * L,), I32),
        pltpu.VMEM_SHARED(((NTILES * CAPB + CH2) * BL,), I32),
        pltpu.VMEM_SHARED((NTILES * L,), I32),
        pltpu.SemaphoreType.DMA, pltpu.SemaphoreType.DMA,
    ]
    return functools.partial(
        pl.kernel, body, mesh=mesh,
        out_type=jax.ShapeDtypeStruct((2 * OUTP, OUTW), F32),
        compiler_params=pltpu.CompilerParams(needs_layout_passes=False),
        scratch_types=scratch)


def kernel(h, c_temp, c_stereo, e_proj, f_Lt, edges, edge_attr, params):
    p = params
    B, N, D = h.shape
    NN = B * N
    E = edges.shape[1]
    h2 = h.reshape(NN, D)

    x = jnp.concatenate(
        [c_temp.reshape(NN, -1), c_stereo.reshape(NN, -1),
         e_proj.reshape(NN, -1), f_Lt.reshape(NN, -1),
         jnp.zeros((NN, 126), F32)], axis=-1)
    w1 = jnp.concatenate([p['W1l'], p['W1r']], axis=1)
    w1 = jnp.concatenate([w1, jnp.zeros((126, 512), F32)], axis=0)
    b1 = jnp.concatenate([p['b1l'], p['b1r']]).reshape(1, 512)
    xlr1 = _mm_bias(x, w1, b1, 1000)
    xl1, xr1 = xlr1[:, :256], xlr1[:, 256:]

    EB = ((E + 255) // 256) * 256
    src = edges[0].astype(I32)
    dst = edges[1].astype(I32)
    eav = edge_attr[:, 0]
    srcp = jnp.concatenate([src, jnp.zeros((EB - E,), I32)])
    dstp = jnp.concatenate([dst, jnp.full((EB - E,), NN, I32)])
    eap = jnp.concatenate([eav, jnp.zeros((EB - E,), F32)])
    ed = jnp.concatenate(
        [srcp.reshape(-1, L), dstp.reshape(-1, L),
         lax.bitcast_convert_type(eap, I32).reshape(-1, L)],
        axis=1).reshape(-1)

    edge1 = _make_edge_call(NN, EB, 256, 64, 2560, 2, 160, 256, True)()
    o1 = edge1(xl1, xr1, ed, p['att1'].reshape(256), p['W1e'][0])
    g1 = jnp.concatenate([o1[0:NN // 2], o1[5120:5120 + NN // 2]])

    w2 = jnp.concatenate([p['W2l'], p['W2r']], axis=1)
    b2 = jnp.concatenate([p['b2l'], p['b2r']]).reshape(1, 2048)
    xlr2 = _relu_mm_bias(g1, p['bias1'].reshape(1, 256), w2, b2, 1000)
    xl2, xr2 = xlr2[:, :1024], xlr2[:, 1024:]

    edge2 = _make_edge_call(NN, EB, 1024, 256, 768, 7, 48, 128, False)()
    o2 = edge2(xl2, xr2, ed, p['att2'].reshape(1024), p['W2e'][0])
    g2 = jnp.concatenate([o2[0:NN // 2], o2[5376:5376 + NN // 2]])

    h_new_flat = _gru(g2, p['bias2'].reshape(1, 256), h2,
                      p['Wih'].T, p['bih'].reshape(1, 768),
                      p['Whh'].T, p['bhh'].reshape(1, 768), 1000)
    h_new = h_new_flat.reshape(B, N, D)

    whid = jnp.concatenate([p['Wr1'], p['Ww1'], p['Wp1'], p['Wd1']], axis=1)
    bhid = jnp.concatenate([p['br1'], p['bw1'], p['bp1'], p['bd1']]).reshape(1, 768)
    w2h = jnp.zeros((512, 4), F32)
    w2h = w2h.at[:256, :2].set(p['Wr2']).at[256:, 2:].set(p['Ww2'])
    b2h = jnp.concatenate([p['br2'], p['bw2']]).reshape(1, 4)
    wpd = jnp.zeros((256, 2), F32)
    wpd = wpd.at[:128, 0].set(p['Wp2'][:, 0]).at[128:, 1].set(p['Wd2'][:, 0])
    bpd = jnp.concatenate([p['bp2'], p['bd2']]).reshape(1, 2)
    rw, pd, ap = _heads(h_new, whid, bhid, w2h, b2h, wpd, bpd)

    return (h_new, rw[..., :2], rw[..., 2:], ap[:, 0, :1], pd[..., 1:2])


# speedup vs baseline: 2.5145x; 1.0297x over previous
import functools

import jax
import jax.numpy as jnp
from jax import lax
from jax.experimental import pallas as pl
from jax.experimental.pallas import tpu as pltpu
from jax.experimental.pallas import tpu_sc as plsc

F32 = jnp.float32
I32 = jnp.int32
L = 16


def _mm_bias_kernel(x_ref, w_ref, b_ref, o_ref):
    o_ref[...] = (
        jnp.dot(x_ref[...], w_ref[...], preferred_element_type=F32) + b_ref[...]
    )


def _mm_bias(x, w, b, bm):
    m, k = x.shape
    n = w.shape[1]
    return pl.pallas_call(
        _mm_bias_kernel,
        grid=(m // bm,),
        in_specs=[
            pl.BlockSpec((bm, k), lambda i: (i, 0)),
            pl.BlockSpec((k, n), lambda i: (0, 0)),
            pl.BlockSpec((1, n), lambda i: (0, 0)),
        ],
        out_specs=pl.BlockSpec((bm, n), lambda i: (i, 0)),
        out_shape=jax.ShapeDtypeStruct((m, n), F32),
    )(x, w, b)


def _relu_mm_bias_kernel(x_ref, b0_ref, w_ref, b_ref, o_ref):
    x1 = jnp.maximum(x_ref[...] + b0_ref[...], 0.0)
    o_ref[...] = jnp.dot(x1, w_ref[...], preferred_element_type=F32) + b_ref[...]


def _relu_mm_bias(x, b0, w, b, bm):
    m, k = x.shape
    n = w.shape[1]
    return pl.pallas_call(
        _relu_mm_bias_kernel,
        grid=(m // bm,),
        in_specs=[
            pl.BlockSpec((bm, k), lambda i: (i, 0)),
            pl.BlockSpec((1, k), lambda i: (0, 0)),
            pl.BlockSpec((k, n), lambda i: (0, 0)),
            pl.BlockSpec((1, n), lambda i: (0, 0)),
        ],
        out_specs=pl.BlockSpec((bm, n), lambda i: (i, 0)),
        out_shape=jax.ShapeDtypeStruct((m, n), F32),
    )(x, b0, w, b)


def _gru_kernel(g2_ref, b2_ref, h_ref, wi_ref, bi_ref, wh_ref, bh_ref, o_ref):
    x2 = g2_ref[...] + b2_ref[...]
    h = h_ref[...]
    gi = jnp.dot(x2, wi_ref[...], preferred_element_type=F32) + bi_ref[...]
    gh = jnp.dot(h, wh_ref[...], preferred_element_type=F32) + bh_ref[...]
    d = h.shape[1]
    ir, iz, inn = gi[:, :d], gi[:, d:2 * d], gi[:, 2 * d:]
    hr, hz, hn = gh[:, :d], gh[:, d:2 * d], gh[:, 2 * d:]
    r = jax.nn.sigmoid(ir + hr)
    z = jax.nn.sigmoid(iz + hz)
    n = jnp.tanh(inn + r * hn)
    o_ref[...] = (1.0 - z) * n + z * h


def _gru(g2, b2, h, wiT, bi, whT, bh, bm):
    m, d = h.shape
    n3 = wiT.shape[1]
    return pl.pallas_call(
        _gru_kernel,
        grid=(m // bm,),
        in_specs=[
            pl.BlockSpec((bm, d), lambda i: (i, 0)),
            pl.BlockSpec((1, d), lambda i: (0, 0)),
            pl.BlockSpec((bm, d), lambda i: (i, 0)),
            pl.BlockSpec((d, n3), lambda i: (0, 0)),
            pl.BlockSpec((1, n3), lambda i: (0, 0)),
            pl.BlockSpec((d, n3), lambda i: (0, 0)),
            pl.BlockSpec((1, n3), lambda i: (0, 0)),
        ],
        out_specs=pl.BlockSpec((bm, d), lambda i: (i, 0)),
        out_shape=jax.ShapeDtypeStruct((m, d), F32),
    )(g2, b2, h, wiT, bi, whT, bh)


def _softplus(v):
    return jnp.maximum(v, 0.0) + jnp.log(1.0 + jnp.exp(-jnp.abs(v)))


def _heads_kernel(h_ref, wh_ref, bh_ref, w2_ref, b2_ref, wpd_ref, bpd_ref,
                  rw_ref, pd_ref, ap_ref):
    h = h_ref[0]
    hid = jnp.maximum(
        jnp.dot(h, wh_ref[...], preferred_element_type=F32) + bh_ref[...], 0.0)
    rw = jnp.dot(hid[:, :512], w2_ref[...], preferred_element_type=F32) + b2_ref[...]
    rw_ref[0] = jnp.concatenate(
        [rw[:, :2], jax.nn.sigmoid(rw[:, 2:])], axis=1)
    pd = _softplus(
        jnp.dot(hid[:, 512:], wpd_ref[...], preferred_element_type=F32)
        + bpd_ref[...])
    pd_ref[0] = pd
    ap_ref[...] = jnp.full((1, 8, 128), jnp.sum(pd[:, 0]) / h.shape[0], F32)


def _heads(h_new, whid, bhid, w2, b2, wpd, bpd):
    b, n, d = h_new.shape
    nh = whid.shape[1]
    return pl.pallas_call(
        _heads_kernel,
        grid=(b,),
        in_specs=[
            pl.BlockSpec((1, n, d), lambda i: (i, 0, 0)),
            pl.BlockSpec((d, nh), lambda i: (0, 0)),
            pl.BlockSpec((1, nh), lambda i: (0, 0)),
            pl.BlockSpec((512, 4), lambda i: (0, 0)),
            pl.BlockSpec((1, 4), lambda i: (0, 0)),
            pl.BlockSpec((256, 2), lambda i: (0, 0)),
            pl.BlockSpec((1, 2), lambda i: (0, 0)),
        ],
        out_specs=[
            pl.BlockSpec((1, n, 4), lambda i: (i, 0, 0)),
            pl.BlockSpec((1, n, 2), lambda i: (i, 0, 0)),
            pl.BlockSpec((1, 8, 128), lambda i: (i, 0, 0)),
        ],
        out_shape=[
            jax.ShapeDtypeStruct((b, n, 4), F32),
            jax.ShapeDtypeStruct((b, n, 2), F32),
            jax.ShapeDtypeStruct((b, 8, 128), F32),
        ],
    )(h_new, whid, bhid, w2, b2, wpd, bpd)


NTILES = 16


def _make_edge_call(NN, EB, K, C, R, NPH, S_t, CAPB, concat):
    H = K // C
    OUTW = 256
    HALF = NN // 2
    OUTP = NPH * R
    EPB = EB // L // NTILES
    BL = 48
    CH2 = 64
    mesh = plsc.VectorSubcoreMesh(core_axis_name="c", subcore_axis_name="s")

    def body(xl_hbm, xr_hbm, ed_hbm, att_hbm, we_hbm, outp_hbm,
             st_buf, cl, ch2, o_src, o_dst, o_ea, att_v, we_v,
             xlbuf, xrbuf, exbuf, acc, den, obuf,
             cntb, cntv, spm_l, spm_c, sem1, sem2):
        cid = lax.axis_index("c")
        tid = lax.axis_index("s")
        lanes = lax.broadcasted_iota(I32, (L,), 0)
        zf = jnp.zeros((L,), F32)
        zi = jnp.zeros((L,), I32)

        pltpu.sync_copy(att_hbm, att_v)
        pltpu.sync_copy(we_hbm, we_v)

        def _phase(p, _):
            lo = cid * HALF + p * R
            hi = cid * HALF + jnp.minimum(p * R + R, HALF)

            def _za(i, _):
                acc[pl.ds(i * L, L)] = zf
                return 0
            lax.fori_loop(0, (S_t + 1) * K // L, _za, 0)
            def _zd(i, _):
                den[pl.ds(i * L, L)] = zf
                return 0
            lax.fori_loop(0, (S_t * 4 + 32) // L, _zd, 0)

            def _s1chunk(ch, cnt):
                pltpu.sync_copy(
                    ed_hbm.at[pl.ds((tid * EPB + ch * 64) * BL, 64 * BL)],
                    st_buf)
                def _scan(i, cnt):
                    sv = st_buf[pl.ds(i * BL, L)]
                    dv = st_buf[pl.ds(i * BL + 16, L)]
                    ev = st_buf[pl.ds(i * BL + 32, L)]
                    m = (dv >= lo) & (dv < hi)
                    mi = m.astype(I32)
                    pos = cnt + plsc.cumsum(mi) - 1
                    pos = jnp.minimum(pos, CAPB * L - 1)
                    posb = lax.shift_right_logical(pos, 4) * BL + (pos & 15)
                    plsc.store_scatter(cl, [posb], sv, mask=m)
                    plsc.store_scatter(cl, [posb + 16], dv, mask=m)
                    plsc.store_scatter(cl, [posb + 32], ev, mask=m)
                    return cnt + jnp.sum(mi)
                return lax.fori_loop(0, 64, _scan, cnt)
            cnt = lax.fori_loop(0, EPB // 64, _s1chunk, jnp.int32(0))

            nblk = lax.shift_right_logical(cnt + 15, 4)
            def _pub(ci, _):
                pltpu.sync_copy(
                    cl.at[pl.ds(ci * 64 * BL, 64 * BL)],
                    spm_l.at[pl.ds((tid * CAPB + ci * 64) * BL, 64 * BL)])
                return 0
            lax.fori_loop(0, lax.shift_right_logical(nblk + 63, 6), _pub, 0)
            cntb[pl.ds(0, L)] = jnp.full((L,), cnt, I32)
            pltpu.sync_copy(cntb, spm_c.at[pl.ds(tid * L, L)])
            plsc.subcore_barrier()

            sub_lo = lo + tid * S_t
            sub_hi = jnp.minimum(sub_lo + S_t, hi)
            pltpu.sync_copy(spm_c, cntv)

            def _batch(jb, _):
                srcv = jnp.clip(o_src[pl.ds(jb * L, L)], 0, NN - 1)
                dstlv = o_dst[pl.ds(jb * L, L)]
                eav = plsc.bitcast(o_ea[pl.ds(jb * L, L)], F32)
                cp1 = pltpu.async_copy(xl_hbm.at[srcv], xlbuf, sem1)
                xrrows = jnp.clip(dstlv + sub_lo, 0, NN - 1)
                cp2 = pltpu.async_copy(xr_hbm.at[xrrows], xrbuf, sem2)
                cp1.wait()
                cp2.wait()
                eax = [eav[ee] for ee in range(L)]
                for h in range(H):
                    def _c(cc, accs):
                        o = h * C + cc * L
                        attv = att_v[pl.ds(o, L)]
                        wev = we_v[pl.ds(o, L)]
                        out = []
                        for e in range(L):
                            u = (xlbuf[e, pl.ds(o, L)]
                                 + xrbuf[e, pl.ds(o, L)] + eax[e] * wev)
                            out.append(
                                accs[e] + jnp.maximum(u, 0.2 * u) * attv)
                        return tuple(out)
                    accs = lax.fori_loop(0, C // L, _c, (zf,) * L)
                    av = zf
                    for e in range(L):
                        av = av + jnp.where(lanes == e, jnp.sum(accs[e]), 0.0)
                    exbuf[pl.ds(h * L, L)] = jnp.exp(av)
                exvs = [exbuf[pl.ds(hh * L, L)] for hh in range(H)]
                for e in range(L):
                    evec = zf
                    for h in range(H):
                        evec = evec + jnp.where(lanes == h, exvs[h][e], 0.0)
                    o4 = dstlv[e] * 4
                    den[pl.ds(o4, L)] = den[pl.ds(o4, L)] + evec
                bases = [dstlv[e] * K for e in range(L)]
                for h in range(H):
                    ss = [exvs[h][e] for e in range(L)]
                    def _p2(cc, _):
                        o = h * C + cc * L
                        xo = pl.ds(o, L)
                        for e in range(L):
                            acc[pl.ds(bases[e] + o, L)] = (
                                acc[pl.ds(bases[e] + o, L)]
                                + ss[e] * xlbuf[e, xo])
                        return 0
                    lax.fori_loop(0, C // L, _p2, 0)
                return 0

            def _j(j, _):
                cntj = cntv[pl.ds(j * L, L)][0]
                nblkj = lax.shift_right_logical(cntj + 15, 4)
                def _chunk(ci, _):
                    pltpu.sync_copy(
                        spm_l.at[pl.ds((j * CAPB + ci * CH2) * BL, CH2 * BL)],
                        ch2)
                    def _scan2(i, c2):
                        gi = ci * CH2 * L + i * L
                        sv = ch2[pl.ds(i * BL, L)]
                        dv = ch2[pl.ds(i * BL + 16, L)]
                        ev = ch2[pl.ds(i * BL + 32, L)]
                        m = ((gi + lanes < cntj) & (dv >= sub_lo)
                             & (dv < sub_hi))
                        mi = m.astype(I32)
                        pos = c2 + plsc.cumsum(mi) - 1
                        plsc.store_scatter(o_src, [pos], sv, mask=m)
                        plsc.store_scatter(o_dst, [pos], dv - sub_lo, mask=m)
                        plsc.store_scatter(o_ea, [pos], ev, mask=m)
                        return c2 + jnp.sum(mi)
                    nsc = jnp.minimum(CH2, nblkj - ci * CH2)
                    c2 = lax.fori_loop(0, nsc, _scan2, jnp.int32(0))
                    o_src[pl.ds(c2, L)] = lanes
                    o_dst[pl.ds(c2, L)] = jnp.full((L,), S_t, I32)
                    o_ea[pl.ds(c2, L)] = zi
                    nb = lax.shift_right_logical(c2 + L - 1, 4)
                    lax.fori_loop(0, nb, _batch, 0)
                    return 0
                lax.fori_loop(
                    0, lax.shift_right_logical(nblkj + CH2 - 1, 6), _chunk, 0)
                return 0
            lax.fori_loop(0, NTILES, _j, 0)

            outbase = cid * OUTP + p * R + tid * S_t
            def _wb(b, _):
                sv0 = 1.0 / (den[pl.ds(b * 32, L)] + 1e-16)
                sv1 = 1.0 / (den[pl.ds(b * 32 + 16, L)] + 1e-16)
                for rr in range(8):
                    sv = sv0 if rr < 4 else sv1
                    ri = rr if rr < 4 else rr - 4
                    for cj in range(OUTW // L):
                        o = cj * L
                        if concat:
                            s = sv[ri * 4 + o // C]
                            obuf[rr, pl.ds(o, L)] = s * acc[
                                pl.ds((b * 8 + rr) * K + o, L)]
                        else:
                            a = zf
                            for h in range(H):
                                s = sv[ri * 4 + h]
                                a = a + s * acc[
                                    pl.ds((b * 8 + rr) * K + h * C + o, L)]
                            obuf[rr, pl.ds(o, L)] = a * (1.0 / H)
                pltpu.sync_copy(obuf, outp_hbm.at[pl.ds(outbase + b * 8, 8)])
                return 0
            lax.fori_loop(0, S_t // 8, _wb, 0)
            plsc.subcore_barrier()
            return 0
        lax.fori_loop(0, NPH, _phase, 0)

    scratch = [
        pltpu.VMEM((64 * BL,), I32),
        pltpu.VMEM((CAPB * BL,), I32),
        pltpu.VMEM((CH2 * BL,), I32),
        pltpu.VMEM((CH2 * L + 48,), I32),
        pltpu.VMEM((CH2 * L + 48,), I32),
        pltpu.VMEM((CH2 * L + 48,), I32),
        pltpu.VMEM((K,), F32), pltpu.VMEM((K,), F32),
        pltpu.VMEM((L, K), F32), pltpu.VMEM((L, K), F32),
        pltpu.VMEM((H * L,), F32),
        pltpu.VMEM(((S_t + 1) * K,), F32),
        pltpu.VMEM((S_t * 4 + 32,), F32),
        pltpu.VMEM((8, OUTW), F32),
        pltpu.VMEM((L,), I32), pltpu.VMEM((NTILES * L,), I32),
        pltpu.VMEM_SHARED(((NTILES * CAPB + CH2) * BL,), I32),
        pltpu.VMEM_SHARED((NTILES * L,), I32),
        pltpu.SemaphoreType.DMA, pltpu.SemaphoreType.DMA,
    ]
    return functools.partial(
        pl.kernel, body, mesh=mesh,
        out_type=jax.ShapeDtypeStruct((2 * OUTP, OUTW), F32),
        compiler_params=pltpu.CompilerParams(needs_layout_passes=False),
        scratch_types=scratch)


def kernel(h, c_temp, c_stereo, e_proj, f_Lt, edges, edge_attr, params):
    p = params
    B, N, D = h.shape
    NN = B * N
    E = edges.shape[1]
    h2 = h.reshape(NN, D)

    x = jnp.concatenate(
        [c_temp.reshape(NN, -1), c_stereo.reshape(NN, -1),
         e_proj.reshape(NN, -1), f_Lt.reshape(NN, -1),
         jnp.zeros((NN, 126), F32)], axis=-1)
    w1 = jnp.concatenate([p['W1l'], p['W1r']], axis=1)
    w1 = jnp.concatenate([w1, jnp.zeros((126, 512), F32)], axis=0)
    b1 = jnp.concatenate([p['b1l'], p['b1r']]).reshape(1, 512)
    xlr1 = _mm_bias(x, w1, b1, 1000)
    xl1, xr1 = xlr1[:, :256], xlr1[:, 256:]

    EB = ((E + 255) // 256) * 256
    src = edges[0].astype(I32)
    dst = edges[1].astype(I32)
    eav = edge_attr[:, 0]
    srcp = jnp.concatenate([src, jnp.zeros((EB - E,), I32)])
    dstp = jnp.concatenate([dst, jnp.full((EB - E,), NN, I32)])
    eap = jnp.concatenate([eav, jnp.zeros((EB - E,), F32)])
    ed = jnp.concatenate(
        [srcp.reshape(-1, L), dstp.reshape(-1, L),
         lax.bitcast_convert_type(eap, I32).reshape(-1, L)],
        axis=1).reshape(-1)

    edge1 = _make_edge_call(NN, EB, 256, 64, 2560, 2, 160, 256, True)()
    o1 = edge1(xl1, xr1, ed, p['att1'].reshape(256), p['W1e'][0])
    g1 = jnp.concatenate([o1[0:NN // 2], o1[5120:5120 + NN // 2]])

    w2 = jnp.concatenate([p['W2l'], p['W2r']], axis=1)
    b2 = jnp.concatenate([p['b2l'], p['b2r']]).reshape(1, 2048)
    xlr2 = _relu_mm_bias(g1, p['bias1'].reshape(1, 256), w2, b2, 1000)
    xl2, xr2 = xlr2[:, :1024], xlr2[:, 1024:]

    edge2 = _make_edge_call(NN, EB, 1024, 256, 768, 7, 48, 128, False)()
    o2 = edge2(xl2, xr2, ed, p['att2'].reshape(1024), p['W2e'][0])
    g2 = jnp.concatenate([o2[0:NN // 2], o2[5376:5376 + NN // 2]])

    h_new_flat = _gru(g2, p['bias2'].reshape(1, 256), h2,
                      p['Wih'].T, p['bih'].reshape(1, 768),
                      p['Whh'].T, p['bhh'].reshape(1, 768), 1000)
    h_new = h_new_flat.reshape(B, N, D)

    whid = jnp.concatenate([p['Wr1'], p['Ww1'], p['Wp1'], p['Wd1']], axis=1)
    bhid = jnp.concatenate([p['br1'], p['bw1'], p['bp1'], p['bd1']]).reshape(1, 768)
    w2h = jnp.zeros((512, 4), F32)
    w2h = w2h.at[:256, :2].set(p['Wr2']).at[256:, 2:].set(p['Ww2'])
    b2h = jnp.concatenate([p['br2'], p['bw2']]).reshape(1, 4)
    wpd = jnp.zeros((256, 2), F32)
    wpd = wpd.at[:128, 0].set(p['Wp2'][:, 0]).at[128:, 1].set(p['Wd2'][:, 0])
    bpd = jnp.concatenate([p['bp2'], p['bd2']]).reshape(1, 2)
    rw, pd, ap = _heads(h_new, whid, bhid, w2h, b2h, wpd, bpd)

    return (h_new, rw[..., :2], rw[..., 2:], ap[:, 0, :1], pd[..., 1:2])
